# baseline (speedup 1.0000x reference)
import functools

import jax
import jax.numpy as jnp
from jax import lax
from jax.experimental import pallas as pl
from jax.experimental.pallas import tpu as pltpu
from jax.experimental.pallas import tpu_sc as plsc

N = 16384
NB = 128
L = 16
NS = 16
NPW = N // NS
NR = NB // L


def _build():
    mesh = plsc.VectorSubcoreMesh(
        core_axis_name="c", subcore_axis_name="s", num_cores=1
    )

    @functools.partial(
        pl.kernel,
        out_type=(
            jax.ShapeDtypeStruct((L,), jnp.float32),
            jax.ShapeDtypeStruct((NS, 2 * NR, L), jnp.float32),
        ),
        mesh=mesh,
        compiler_params=pltpu.CompilerParams(needs_layout_passes=False),
        scratch_types=[
            pltpu.VMEM((NPW,), jnp.float32),
            pltpu.VMEM((NPW,), jnp.float32),
            pltpu.VMEM((NPW,), jnp.int32),
            pltpu.VMEM((NB,), jnp.float32),
            pltpu.VMEM((NB,), jnp.float32),
            pltpu.VMEM((2 * NR, L), jnp.float32),
            pltpu.VMEM((NS, 2 * NR, L), jnp.float32),
            pltpu.VMEM((L,), jnp.float32),
        ],
    )
    def k(o_hbm, l_hbm, id_hbm, out_hbm, scr_hbm,
          o_v, l_v, id_v, acc, cnt, pak, buf, ov):
        s = lax.axis_index("s")
        base = s * NPW
        pltpu.sync_copy(o_hbm.at[pl.ds(base, NPW)], o_v)
        pltpu.sync_copy(l_hbm.at[pl.ds(base, NPW)], l_v)
        pltpu.sync_copy(id_hbm.at[pl.ds(base, NPW)], id_v)

        zeros = jnp.zeros((L,), jnp.float32)
        ones = jnp.ones((L,), jnp.float32)

        def zero_body(j, carry):
            acc[pl.ds(j * L, L)] = zeros
            cnt[pl.ds(j * L, L)] = zeros
            return carry

        lax.fori_loop(0, NR, zero_body, 0)

        def accum_body(i, carry):
            o = o_v[pl.ds(i * L, L)]
            t = l_v[pl.ds(i * L, L)]
            idx = id_v[pl.ds(i * L, L)]
            d = o - t
            plsc.addupdate_scatter(acc, [idx], d * d)
            plsc.addupdate_scatter(cnt, [idx], ones)
            return carry

        lax.fori_loop(0, NPW // L, accum_body, 0)

        def pack_body(j, carry):
            pak[j, ...] = acc[pl.ds(j * L, L)]
            pak[NR + j, ...] = cnt[pl.ds(j * L, L)]
            return carry

        lax.fori_loop(0, NR, pack_body, 0)

        pltpu.sync_copy(pak, scr_hbm.at[s])
        plsc.subcore_barrier()

        @pl.when(s == 0)
        def _():
            pltpu.sync_copy(scr_hbm, buf)

            def chunk_body(j, carry):
                sum_mse, ndom = carry

                def row_body(r, c2):
                    ta, tc = c2
                    ta = ta + buf[r, j, ...]
                    tc = tc + buf[r, NR + j, ...]
                    return ta, tc

                ta, tc = lax.fori_loop(0, NS, row_body, (zeros, zeros))
                present = tc > 0.0
                safe = jnp.where(present, tc, ones)
                sum_mse = sum_mse + jnp.where(present, ta / safe, zeros)
                ndom = ndom + jnp.where(present, ones, zeros)
                return sum_mse, ndom

            sum_mse, ndom = lax.fori_loop(0, NR, chunk_body, (zeros, zeros))
            total = jnp.full((L,), jnp.sum(sum_mse), jnp.float32)
            nd = jnp.full((L,), jnp.sum(ndom), jnp.float32)
            ov[...] = total / nd
            pltpu.sync_copy(ov, out_hbm)

    return k


_KERNEL = _build()


@jax.jit
def kernel(outputs, labels, domain_ids):
    res, _ = _KERNEL(outputs, labels, domain_ids.astype(jnp.int32))
    return res[0]


# speedup vs baseline: 1.0473x; 1.0473x over previous
import functools

import jax
import jax.numpy as jnp
from jax import lax
from jax.experimental import pallas as pl
from jax.experimental.pallas import tpu as pltpu
from jax.experimental.pallas import tpu_sc as plsc

N = 16384
NB = 128
L = 16
NS = 16
NPW = N // NS
NR = NB // L


def _build():
    mesh = plsc.VectorSubcoreMesh(
        core_axis_name="c", subcore_axis_name="s", num_cores=1
    )

    @functools.partial(
        pl.kernel,
        out_type=(
            jax.ShapeDtypeStruct((L,), jnp.float32),
            jax.ShapeDtypeStruct((NS, 2 * NR, L), jnp.float32),
        ),
        mesh=mesh,
        compiler_params=pltpu.CompilerParams(needs_layout_passes=False),
        scratch_types=[
            pltpu.VMEM((NPW,), jnp.float32),
            pltpu.VMEM((NPW,), jnp.float32),
            pltpu.VMEM((NPW,), jnp.int32),
            pltpu.VMEM((NB,), jnp.float32),
            pltpu.VMEM((NB,), jnp.float32),
            pltpu.VMEM((2 * NR, L), jnp.float32),
            pltpu.VMEM((NS, 2 * NR, L), jnp.float32),
            pltpu.VMEM((L,), jnp.float32),
            pltpu.SemaphoreType.DMA,
        ],
    )
    def k(o_hbm, l_hbm, id_hbm, out_hbm, scr_hbm,
          o_v, l_v, id_v, acc, cnt, pak, buf, ov, sem):
        s = lax.axis_index("s")
        base = s * NPW
        co = pltpu.async_copy(o_hbm.at[pl.ds(base, NPW)], o_v, sem)
        cl = pltpu.async_copy(l_hbm.at[pl.ds(base, NPW)], l_v, sem)
        ci = pltpu.async_copy(id_hbm.at[pl.ds(base, NPW)], id_v, sem)

        zeros = jnp.zeros((L,), jnp.float32)
        ones = jnp.ones((L,), jnp.float32)

        def zero_body(j, carry):
            acc[pl.ds(j * L, L)] = zeros
            cnt[pl.ds(j * L, L)] = zeros
            return carry

        lax.fori_loop(0, NR, zero_body, 0)

        co.wait()
        cl.wait()
        ci.wait()

        def accum_body(i, carry):
            o = o_v[pl.ds(i * L, L)]
            t = l_v[pl.ds(i * L, L)]
            idx = id_v[pl.ds(i * L, L)]
            d = o - t
            plsc.addupdate_scatter(acc, [idx], d * d)
            plsc.addupdate_scatter(cnt, [idx], ones)
            return carry

        lax.fori_loop(0, NPW // L, accum_body, 0)

        def pack_body(j, carry):
            pak[j, ...] = acc[pl.ds(j * L, L)]
            pak[NR + j, ...] = cnt[pl.ds(j * L, L)]
            return carry

        lax.fori_loop(0, NR, pack_body, 0)

        pltpu.sync_copy(pak, scr_hbm.at[s])
        plsc.subcore_barrier()

        @pl.when(s == 0)
        def _():
            pltpu.sync_copy(scr_hbm, buf)

            def chunk_body(j, carry):
                sum_mse, ndom = carry

                def row_body(r, c2):
                    ta, tc = c2
                    ta = ta + buf[r, j, ...]
                    tc = tc + buf[r, NR + j, ...]
                    return ta, tc

                ta, tc = lax.fori_loop(0, NS, row_body, (zeros, zeros))
                present = tc > 0.0
                safe = jnp.where(present, tc, ones)
                sum_mse = sum_mse + jnp.where(present, ta / safe, zeros)
                ndom = ndom + jnp.where(present, ones, zeros)
                return sum_mse, ndom

            sum_mse, ndom = lax.fori_loop(0, NR, chunk_body, (zeros, zeros))
            total = jnp.full((L,), jnp.sum(sum_mse), jnp.float32)
            nd = jnp.full((L,), jnp.sum(ndom), jnp.float32)
            ov[...] = total / nd
            pltpu.sync_copy(ov, out_hbm)

    return k


_KERNEL = _build()


@jax.jit
def kernel(outputs, labels, domain_ids):
    res, _ = _KERNEL(outputs, labels, domain_ids.astype(jnp.int32))
    return res[0]
